# baseline (speedup 1.0000x reference)
import jax
import jax.numpy as jnp
from jax import lax
from jax.experimental import pallas as pl
from jax.experimental.pallas import tpu as pltpu
from jax.experimental.pallas import tpu_sc as plsc

BATCH = 4
SEQ = 8192
D = 768
LANES = 16

N_ROWS = BATCH * SEQ
NC, NS = 2, 16
NW = NC * NS
ROWS_PER_W = N_ROWS // NW
CHUNK = 32
N_CHUNKS = ROWS_PER_W // CHUNK
NSLOTS = 2
N_GROUPS = N_CHUNKS // NSLOTS
DW = D // 2
G_VECS = DW // LANES


def _body(x_hbm, idx_hbm, table_hbm, out_hbm, idx_v,
          emb_s, x_s, gsems, xsems, osems):
    wid = lax.axis_index("s") * NC + lax.axis_index("c")
    base = wid * ROWS_PER_W

    pltpu.sync_copy(idx_hbm.at[pl.ds(base, ROWS_PER_W)], idx_v)

    def start(c, s):
        pltpu.async_copy(
            table_hbm.at[idx_v.at[pl.ds(c * CHUNK, CHUNK)]], emb_s[s],
            gsems[s])
        pltpu.async_copy(x_hbm.at[pl.ds(base + c * CHUNK, CHUNK)], x_s[s],
                         xsems[s])

    def wait(c, s):
        pltpu.make_async_copy(
            table_hbm.at[idx_v.at[pl.ds(c * CHUNK, CHUNK)]], emb_s[s],
            gsems[s]).wait()
        pltpu.make_async_copy(
            x_hbm.at[pl.ds(base + c * CHUNK, CHUNK)], x_s[s],
            xsems[s]).wait()

    def add_chunk(s):
        mask = jnp.int32(-65536)

        def add_row(i, _):
            for g in range(G_VECS):
                v = emb_s[s][i, pl.ds(g * LANES, LANES)]
                lo = lax.bitcast_convert_type(v << 16, jnp.float32)
                hi = lax.bitcast_convert_type(v & mask, jnp.float32)
                plsc.addupdate(x_s[s].at[i, pl.ds(2 * g * LANES, LANES)], lo)
                plsc.addupdate(
                    x_s[s].at[i, pl.ds((2 * g + 1) * LANES, LANES)], hi)
            return 0

        lax.fori_loop(0, CHUNK, add_row, 0)

    def out_start(c, s):
        pltpu.async_copy(x_s[s], out_hbm.at[pl.ds(base + c * CHUNK, CHUNK)],
                         osems[s])

    def out_wait(c, s):
        pltpu.make_async_copy(
            x_s[s], out_hbm.at[pl.ds(base + c * CHUNK, CHUNK)],
            osems[s]).wait()

    for s in range(NSLOTS):
        start(s, s)

    def group_step(i, _):
        c0 = i * NSLOTS
        for s in range(NSLOTS):
            wait(c0 + s, s)
            add_chunk(s)
            out_start(c0 + s, s)
        for s in range(NSLOTS):
            out_wait(c0 + s, s)
            start(c0 + NSLOTS + s, s)
        return 0

    lax.fori_loop(0, N_GROUPS - 1, group_step, 0)

    c0 = (N_GROUPS - 1) * NSLOTS
    for s in range(NSLOTS):
        wait(c0 + s, s)
        add_chunk(s)
        out_start(c0 + s, s)
    for s in range(NSLOTS):
        out_wait(c0 + s, s)


@jax.jit
def _run(x2d, idx, table_i32):
    mesh = plsc.VectorSubcoreMesh(core_axis_name="c", subcore_axis_name="s")
    return pl.kernel(
        _body,
        out_type=jax.ShapeDtypeStruct((N_ROWS, D), jnp.float32),
        mesh=mesh,
        scratch_types=[
            pltpu.VMEM((ROWS_PER_W,), jnp.int32),
            [pltpu.VMEM((CHUNK, DW), jnp.int32) for _ in range(NSLOTS)],
            [pltpu.VMEM((CHUNK, D), jnp.float32) for _ in range(NSLOTS)],
            [pltpu.SemaphoreType.DMA for _ in range(NSLOTS)],
            [pltpu.SemaphoreType.DMA for _ in range(NSLOTS)],
            [pltpu.SemaphoreType.DMA for _ in range(NSLOTS)],
        ],
    )(x2d, idx, table_i32)


def kernel(x, position_ids, embeddings):
    x2d = x.reshape(N_ROWS, D)
    idx = position_ids.astype(jnp.int32).reshape(N_ROWS)
    tb = embeddings.astype(jnp.bfloat16).reshape(-1, D // 32, 2, LANES)
    tb = tb.transpose(0, 1, 3, 2)
    table_i32 = jax.lax.bitcast_convert_type(tb, jnp.int32).reshape(-1, DW)
    out = _run(x2d, idx, table_i32)
    return out.reshape(BATCH, SEQ, D)


# speedup vs baseline: 1.7277x; 1.7277x over previous
import jax
import jax.numpy as jnp
from jax import lax
from jax.experimental import pallas as pl
from jax.experimental.pallas import tpu as pltpu
from jax.experimental.pallas import tpu_sc as plsc

BATCH = 4
SEQ = 8192
D = 768
LANES = 16

N_ROWS = BATCH * SEQ
NC, NS = 2, 16
NW = NC * NS
ROWS_PER_W = N_ROWS // NW
CHUNK = 32
N_CHUNKS = ROWS_PER_W // CHUNK
NSLOTS = 2
N_GROUPS = N_CHUNKS // NSLOTS
D_VECS = D // LANES


def _body(x_hbm, idx_hbm, table_hbm, out_hbm, idx_v,
          rows_s, x_s, gsems, xsems, osems):
    wid = lax.axis_index("s") * NC + lax.axis_index("c")
    base = wid * ROWS_PER_W

    pltpu.sync_copy(idx_hbm.at[pl.ds(base, ROWS_PER_W)], idx_v)

    def g_start(c, s):
        pltpu.async_copy(
            table_hbm.at[idx_v.at[pl.ds(c * CHUNK, CHUNK)]], rows_s[s],
            gsems[s])

    def x_start(c, s):
        pltpu.async_copy(x_hbm.at[pl.ds(base + c * CHUNK, CHUNK)], x_s[s],
                         xsems[s])

    def wait(c, s):
        pltpu.make_async_copy(
            table_hbm.at[idx_v.at[pl.ds(c * CHUNK, CHUNK)]], rows_s[s],
            gsems[s]).wait()
        pltpu.make_async_copy(
            x_hbm.at[pl.ds(base + c * CHUNK, CHUNK)], x_s[s],
            xsems[s]).wait()

    def add_chunk(s):
        def add_row(i, _):
            for j in range(D_VECS):
                sl = pl.ds(j * LANES, LANES)
                plsc.addupdate(x_s[s].at[i, sl], rows_s[s][i, sl])
            return 0

        lax.fori_loop(0, CHUNK, add_row, 0)

    def out_start(c, s):
        pltpu.async_copy(x_s[s], out_hbm.at[pl.ds(base + c * CHUNK, CHUNK)],
                         osems[s])

    def out_wait(c, s):
        pltpu.make_async_copy(
            x_s[s], out_hbm.at[pl.ds(base + c * CHUNK, CHUNK)],
            osems[s]).wait()

    for s in range(NSLOTS):
        g_start(s, s)
        x_start(s, s)

    def group_step(i, _):
        c0 = i * NSLOTS
        for s in range(NSLOTS):
            wait(c0 + s, s)
            add_chunk(s)
            g_start(c0 + NSLOTS + s, s)
            out_start(c0 + s, s)
        for s in range(NSLOTS):
            out_wait(c0 + s, s)
            x_start(c0 + NSLOTS + s, s)
        return 0

    lax.fori_loop(0, N_GROUPS - 1, group_step, 0)

    c0 = (N_GROUPS - 1) * NSLOTS
    for s in range(NSLOTS):
        wait(c0 + s, s)
        add_chunk(s)
        out_start(c0 + s, s)
    for s in range(NSLOTS):
        out_wait(c0 + s, s)


@jax.jit
def _run(x2d, idx, table):
    mesh = plsc.VectorSubcoreMesh(core_axis_name="c", subcore_axis_name="s")
    return pl.kernel(
        _body,
        out_type=jax.ShapeDtypeStruct((N_ROWS, D), jnp.float32),
        mesh=mesh,
        scratch_types=[
            pltpu.VMEM((ROWS_PER_W,), jnp.int32),
            [pltpu.VMEM((CHUNK, D), jnp.float32) for _ in range(NSLOTS)],
            [pltpu.VMEM((CHUNK, D), jnp.float32) for _ in range(NSLOTS)],
            [pltpu.SemaphoreType.DMA for _ in range(NSLOTS)],
            [pltpu.SemaphoreType.DMA for _ in range(NSLOTS)],
            [pltpu.SemaphoreType.DMA for _ in range(NSLOTS)],
        ],
    )(x2d, idx, table)


def kernel(x, position_ids, embeddings):
    x2d = x.reshape(N_ROWS, D)
    idx = position_ids.astype(jnp.int32).reshape(N_ROWS)
    out = _run(x2d, idx, embeddings)
    return out.reshape(BATCH, SEQ, D)
